# baseline (speedup 1.0000x reference)
import functools

import jax
import jax.numpy as jnp
from jax import lax
from jax.experimental import pallas as pl
from jax.experimental.pallas import tpu as pltpu
from jax.experimental.pallas import tpu_sc as plsc

_FIELD0_ROWS = 1_000_000
_BATCH = 16384
_D = 16

_info = plsc.get_sparse_core_info()
_NC, _NS, _L = _info.num_cores, _info.num_subcores, _info.num_lanes
_NW = _NC * _NS
_BPW = _BATCH // _NW
_CHUNK = 128
_NCHUNK = _BPW // _CHUNK
_GRP = _CHUNK // _L


def _gmf_body(x0_hbm, x1_hbm, tbl_hbm, w_hbm, b_hbm, out_hbm,
              xv0, xv1, gi0, gi1, r0a, r0b, r1a, r1b, w_v, b_v, out_v,
              sem_a, sem_b):
    wid = lax.axis_index("s") * _NC + lax.axis_index("c")
    base = wid * _BPW

    pltpu.sync_copy(x0_hbm.at[pl.ds(base, _BPW)], xv0)
    pltpu.sync_copy(x1_hbm.at[pl.ds(base, _BPW)], xv1)
    pltpu.sync_copy(w_hbm, w_v)
    pltpu.sync_copy(b_hbm, b_v)

    for c in range(_NCHUNK):
        for k in range(_GRP):
            sl = pl.ds(k * _L, _L)
            fl = pl.ds(c * _CHUNK + k * _L, _L)
            gi0[c, sl] = lax.shift_right_logical(xv0[fl], 3)
            gi1[c, sl] = lax.shift_right_logical(xv1[fl] + _FIELD0_ROWS, 3)

    rbuf0 = (r0a, r0b)
    rbuf1 = (r1a, r1b)
    sems = (sem_a, sem_b)

    def fire(c):
        s = c % 2
        return (pltpu.async_copy(tbl_hbm.at[gi0.at[c]], rbuf0[s], sems[s]),
                pltpu.async_copy(tbl_hbm.at[gi1.at[c]], rbuf1[s], sems[s]))

    w = w_v[...]
    bv = b_v[...]
    lane = lax.iota(jnp.int32, _L)
    wd = [lax.gather(w, jnp.full((_L, 1), d, jnp.int32),
                     lax.GatherDimensionNumbers(
                         offset_dims=(), collapsed_slice_dims=(0,),
                         start_index_map=(0,)),
                     (1,), mode=lax.GatherScatterMode.PROMISE_IN_BOUNDS)
          for d in range(_D)]

    inflight = fire(0)
    for c in range(_NCHUNK):
        for cp in inflight:
            cp.wait()
        if c + 1 < _NCHUNK:
            inflight = fire(c + 1)
        r0 = rbuf0[c % 2]
        r1 = rbuf1[c % 2]

        def group(g, _, c=c, r0=r0, r1=r1):
            fl = pl.ds(c * _CHUNK + g * _L, _L)
            row = g * _L + lane
            off0 = lax.shift_left(jnp.bitwise_and(xv0[fl], 7), 4)
            off1 = lax.shift_left(
                jnp.bitwise_and(xv1[fl] + _FIELD0_ROWS, 7), 4)
            acc = bv
            for d in range(_D):
                u = plsc.load_gather(r0, [row, off0 + d])
                v = plsc.load_gather(r1, [row, off1 + d])
                acc = acc + u * v * wd[d]
            out_v[fl] = jnp.maximum(acc, 0.0)
            return 0

        lax.fori_loop(0, _GRP, group, 0)

    pltpu.sync_copy(out_v, out_hbm.at[pl.ds(base, _BPW)])


@jax.jit
def _gmf(x0, x1, tbl, w, bvec):
    mesh = plsc.VectorSubcoreMesh(core_axis_name="c", subcore_axis_name="s")
    run = functools.partial(
        pl.kernel,
        mesh=mesh,
        compiler_params=pltpu.CompilerParams(needs_layout_passes=False),
        out_type=jax.ShapeDtypeStruct((_BATCH,), jnp.float32),
        scratch_types=[
            pltpu.VMEM((_BPW,), jnp.int32),
            pltpu.VMEM((_BPW,), jnp.int32),
            pltpu.VMEM((_NCHUNK, _CHUNK), jnp.int32),
            pltpu.VMEM((_NCHUNK, _CHUNK), jnp.int32),
            pltpu.VMEM((_CHUNK, 8 * _D), jnp.float32),
            pltpu.VMEM((_CHUNK, 8 * _D), jnp.float32),
            pltpu.VMEM((_CHUNK, 8 * _D), jnp.float32),
            pltpu.VMEM((_CHUNK, 8 * _D), jnp.float32),
            pltpu.VMEM((_D,), jnp.float32),
            pltpu.VMEM((_L,), jnp.float32),
            pltpu.VMEM((_BPW,), jnp.float32),
            pltpu.SemaphoreType.DMA,
            pltpu.SemaphoreType.DMA,
        ],
    )(_gmf_body)
    return run(x0, x1, tbl, w, bvec)


def kernel(x, table, W, b):
    x0 = x[:, 0].astype(jnp.int32)
    x1 = x[:, 1].astype(jnp.int32)
    tbl = table.reshape(2_000_000 * _D // 128, 128)
    w = W.reshape(_D)
    bvec = jnp.broadcast_to(b.reshape(()), (_L,)).astype(jnp.float32)
    y = _gmf(x0, x1, tbl, w, bvec)
    return y.reshape(_BATCH, 1)
